# baseline (speedup 1.0000x reference)
import functools

import jax
import jax.numpy as jnp
from jax import lax
from jax.experimental import pallas as pl
from jax.experimental.pallas import tpu as pltpu
from jax.experimental.pallas import tpu_sc as plsc

DIM = 64
B = 16384
LAMBD = 0.1
NW = 32
CHUNK = B // NW
SEG = 128
NSEG = CHUNK // SEG
BLK = 1024
GRID = B // BLK

_mesh = plsc.VectorSubcoreMesh(core_axis_name="c", subcore_axis_name="s",
                               num_cores=2, num_subcores=16)


def _make_gather(rows):
    @functools.partial(
        pl.kernel,
        mesh=_mesh,
        compiler_params=pltpu.CompilerParams(needs_layout_passes=False,
                                             use_tc_tiling_on_sc=True,
                                             skip_device_barrier=True),
        out_type=jax.ShapeDtypeStruct((B, 2 * DIM), jnp.float32),
        scratch_types=[
            pltpu.VMEM((NSEG, SEG), jnp.int32),
            pltpu.VMEM((CHUNK, 2 * DIM), jnp.float32),
            pltpu.SemaphoreType.DMA,
        ],
        name=f"sc_gather_{rows}",
    )
    def gather(idx_hbm, tab_hbm, out_hbm, idx_v, rows_v, sem):
        wid = lax.axis_index("s") * 2 + lax.axis_index("c")
        base = wid * CHUNK
        pltpu.sync_copy(idx_hbm.at[pl.ds(wid * NSEG, NSEG)], idx_v)
        handles = [
            pltpu.async_copy(tab_hbm.at[idx_v.at[k]],
                             rows_v.at[pl.ds(k * SEG, SEG)], sem)
            for k in range(NSEG)
        ]
        for h in handles:
            h.wait()
        pltpu.sync_copy(rows_v, out_hbm.at[pl.ds(base, CHUNK)])

    return gather


_gather_node = _make_gather(500000)
_gather_base = _make_gather(50000)


def _half_dots(x, y):
    xl, xr = x[:, :DIM], x[:, DIM:]
    yl, yr = y[:, :DIM], y[:, DIM:]
    return (jnp.sum(xl * yl, axis=1, keepdims=True),
            jnp.sum(xl * yr, axis=1, keepdims=True),
            jnp.sum(xr * yl, axis=1, keepdims=True),
            jnp.sum(xr * yr, axis=1, keepdims=True))


def _select4(ll, lr, rl, rr, pa, pb):
    qa, qb = 1.0 - pa, 1.0 - pb
    return qa * qb * ll + qa * pb * lr + pa * qb * rl + pa * pb * rr


def _loss_body(a_ref, b_ref, c_ref, d_ref,
               pa_ref, pb_ref, pc_ref, pd_ref, t_ref,
               o_ref, acc_ref):
    g = pl.program_id(0)

    a, b = a_ref[...], b_ref[...]
    pa, pb = pa_ref[...], pb_ref[...]
    sll, slr, srl, srr = _half_dots(a, b)
    s = _select4(sll, slr, srl, srr, pa, pb)
    al, ar = a[:, :DIM], a[:, DIM:]
    bl, br = b[:, :DIM], b[:, DIM:]
    na = _select4(jnp.sum(al * al, axis=1, keepdims=True), 0.0, 0.0,
                  jnp.sum(ar * ar, axis=1, keepdims=True), pa, pa)
    nb = _select4(jnp.sum(bl * bl, axis=1, keepdims=True), 0.0, 0.0,
                  jnp.sum(br * br, axis=1, keepdims=True), pb, pb)
    na = jnp.maximum(jnp.sqrt(na), 1e-12)
    nb = jnp.maximum(jnp.sqrt(nb), 1e-12)
    scores = jax.nn.sigmoid(s / (na * nb))
    t = t_ref[...]
    ml = t * jnp.log(scores) + (1.0 - t) * jnp.log(1.0 - scores)

    c, d = c_ref[...], d_ref[...]
    rll, rlr, rrl, rrr = _half_dots(c, d)
    r = _select4(rll, rlr, rrl, rrr, pc_ref[...], pd_ref[...])
    rl_ = jnp.log(jax.nn.sigmoid(jnp.clip(r, -15.0, 15.0)))

    partial = jnp.sum(ml) + LAMBD * jnp.sum(rl_)

    @pl.when(g == 0)
    def _():
        acc_ref[0, 0] = 0.0

    acc_ref[0, 0] += partial

    @pl.when(g == GRID - 1)
    def _():
        o_ref[...] = jnp.reshape(-acc_ref[0, 0] / B, (1, 1))


_finish = pl.pallas_call(
    _loss_body,
    grid=(GRID,),
    in_specs=[pl.BlockSpec((BLK, 2 * DIM), lambda g: (g, 0))] * 4
    + [pl.BlockSpec((BLK, 1), lambda g: (g, 0))] * 5,
    out_specs=pl.BlockSpec((1, 1), lambda g: (0, 0)),
    out_shape=jax.ShapeDtypeStruct((1, 1), jnp.float32),
    scratch_shapes=[pltpu.SMEM((1, 1), jnp.float32)],
)


def _split(idx):
    idx = idx.astype(jnp.int32)
    phys = (idx >> 1).reshape(NW * NSEG, SEG)
    par = (idx & 1).astype(jnp.float32).reshape(B, 1)
    return phys, par


@jax.jit
def kernel(sources, contexts, targets, personas, pure_sources,
           node_embedding, node_noise_embedding, base_node_embedding):
    srcp, pa = _split(sources)
    ctxp, pb = _split(contexts)
    purep, pc = _split(pure_sources)
    perp, pd = _split(personas)
    node2 = node_embedding.reshape(-1, 2 * DIM)
    noise2 = node_noise_embedding.reshape(-1, 2 * DIM)
    base2 = base_node_embedding.reshape(-1, 2 * DIM)
    rows_src = _gather_node(srcp, node2)
    rows_ctx = _gather_node(ctxp, noise2)
    rows_pure = _gather_node(purep, node2)
    rows_per = _gather_base(perp, base2)
    out = _finish(rows_src, rows_ctx, rows_pure, rows_per,
                  pa, pb, pc, pd, targets.reshape(B, 1))
    return out.reshape(())


# speedup vs baseline: 1.0207x; 1.0207x over previous
import functools

import jax
import jax.numpy as jnp
from jax import lax
from jax.experimental import pallas as pl
from jax.experimental.pallas import tpu as pltpu
from jax.experimental.pallas import tpu_sc as plsc

DIM = 64
B = 16384
LAMBD = 0.1
NW = 32
CHUNK = B // NW
SEG = 128
NSEG = CHUNK // SEG
SUB = 2
ROWS = CHUNK // SUB
SEG_PER_SUB = ROWS // SEG
GROUPS = ROWS // 16

_mesh = plsc.VectorSubcoreMesh(core_axis_name="c", subcore_axis_name="s",
                               num_cores=2, num_subcores=16)


@functools.partial(
    pl.kernel,
    mesh=_mesh,
    compiler_params=pltpu.CompilerParams(needs_layout_passes=False,
                                         use_tc_tiling_on_sc=False,
                                         skip_device_barrier=True),
    out_type=[
        jax.ShapeDtypeStruct((B,), jnp.float32),
        jax.ShapeDtypeStruct((B,), jnp.float32),
        jax.ShapeDtypeStruct((B,), jnp.float32),
        jax.ShapeDtypeStruct((B,), jnp.float32),
    ],
    scratch_types=[
        pltpu.VMEM((NSEG, SEG), jnp.int32),
        pltpu.VMEM((NSEG, SEG), jnp.int32),
        pltpu.VMEM((CHUNK,), jnp.int32),
        pltpu.VMEM((CHUNK,), jnp.int32),
        pltpu.VMEM((ROWS, 2 * DIM), jnp.float32),
        pltpu.VMEM((ROWS, 2 * DIM), jnp.float32),
        pltpu.VMEM((CHUNK,), jnp.float32),
        pltpu.VMEM((CHUNK,), jnp.float32),
        pltpu.VMEM((CHUNK,), jnp.float32),
        pltpu.SemaphoreType.DMA,
    ],
)
def _sc_gather_dot(srcp_hbm, srch_hbm, ctxp_hbm, ctxh_hbm,
                   purep_hbm, pureh_hbm, perp_hbm, perh_hbm,
                   node_hbm, noise_hbm, base_hbm,
                   s_out, na_out, nb_out, r_out,
                   idx_a, idx_b, ho_a, ho_b, rows_a, rows_b,
                   s_v, na_v, nb_v, sem):
    wid = lax.axis_index("s") * 2 + lax.axis_index("c")
    base = wid * CHUNK

    def gather_sub(tab_a, tab_b, sc):
        handles = []
        for i in range(SEG_PER_SUB):
            k = sc * SEG_PER_SUB + i
            handles.append(pltpu.async_copy(
                tab_a.at[idx_a.at[k]], rows_a.at[pl.ds(i * SEG, SEG)], sem))
            handles.append(pltpu.async_copy(
                tab_b.at[idx_b.at[k]], rows_b.at[pl.ds(i * SEG, SEG)], sem))
        return handles

    def drain(handles):
        for h in handles:
            h.wait()

    zero = jnp.zeros((16,), jnp.float32)
    iota16 = lax.iota(jnp.int32, 16)

    pltpu.sync_copy(srcp_hbm.at[wid], idx_a)
    pltpu.sync_copy(ctxp_hbm.at[wid], idx_b)
    pltpu.sync_copy(srch_hbm.at[wid], ho_a)
    pltpu.sync_copy(ctxh_hbm.at[wid], ho_b)

    for sc in range(SUB):
        drain(gather_sub(node_hbm, noise_hbm, sc))

        def main_group(g, _):
            rows = g * 16 + iota16
            off = sc * ROWS + g * 16
            ca0 = ho_a[pl.ds(off, 16)]
            cb0 = ho_b[pl.ds(off, 16)]

            def col(j, acc):
                s, na, nb = acc
                a = plsc.load_gather(rows_a, [rows, ca0 + j])
                b = plsc.load_gather(rows_b, [rows, cb0 + j])
                return (s + a * b, na + a * a, nb + b * b)

            s, na, nb = lax.fori_loop(0, DIM, col, (zero, zero, zero))
            s_v[pl.ds(off, 16)] = s
            na_v[pl.ds(off, 16)] = na
            nb_v[pl.ds(off, 16)] = nb
            return 0

        lax.fori_loop(0, GROUPS, main_group, 0)

    pltpu.sync_copy(s_v, s_out.at[pl.ds(base, CHUNK)])
    pltpu.sync_copy(na_v, na_out.at[pl.ds(base, CHUNK)])
    pltpu.sync_copy(nb_v, nb_out.at[pl.ds(base, CHUNK)])

    pltpu.sync_copy(purep_hbm.at[wid], idx_a)
    pltpu.sync_copy(perp_hbm.at[wid], idx_b)
    pltpu.sync_copy(pureh_hbm.at[wid], ho_a)
    pltpu.sync_copy(perh_hbm.at[wid], ho_b)

    for sc in range(SUB):
        drain(gather_sub(node_hbm, base_hbm, sc))

        def reg_group(g, _):
            rows = g * 16 + iota16
            off = sc * ROWS + g * 16
            ca0 = ho_a[pl.ds(off, 16)]
            cb0 = ho_b[pl.ds(off, 16)]

            def col(j, s):
                a = plsc.load_gather(rows_a, [rows, ca0 + j])
                b = plsc.load_gather(rows_b, [rows, cb0 + j])
                return s + a * b

            s = lax.fori_loop(0, DIM, col, zero)
            s_v[pl.ds(off, 16)] = s
            return 0

        lax.fori_loop(0, GROUPS, reg_group, 0)

    pltpu.sync_copy(s_v, r_out.at[pl.ds(base, CHUNK)])


def _finish_body(t_ref, s_ref, na_ref, nb_ref, r_ref, o_ref):
    na = jnp.maximum(jnp.sqrt(na_ref[...]), 1e-12)
    nb = jnp.maximum(jnp.sqrt(nb_ref[...]), 1e-12)
    scores = jax.nn.sigmoid(s_ref[...] / (na * nb))
    t = t_ref[...]
    main = t * jnp.log(scores) + (1.0 - t) * jnp.log(1.0 - scores)
    main_loss = -jnp.mean(main)
    r = jax.nn.sigmoid(jnp.clip(r_ref[...], -15.0, 15.0))
    reg_loss = -jnp.mean(jnp.log(r))
    o_ref[...] = jnp.reshape(main_loss + LAMBD * reg_loss, (1, 1))


_finish = pl.pallas_call(
    _finish_body,
    out_shape=jax.ShapeDtypeStruct((1, 1), jnp.float32),
)


def _split_idx(idx):
    idx = idx.astype(jnp.int32)
    phys = (idx >> 1).reshape(NW, NSEG, SEG)
    half = ((idx & 1) * DIM).reshape(NW, CHUNK)
    return phys, half


@jax.jit
def kernel(sources, contexts, targets, personas, pure_sources,
           node_embedding, node_noise_embedding, base_node_embedding):
    srcp, srch = _split_idx(sources)
    ctxp, ctxh = _split_idx(contexts)
    purep, pureh = _split_idx(pure_sources)
    perp, perh = _split_idx(personas)
    node2 = node_embedding.reshape(-1, 2 * DIM)
    noise2 = node_noise_embedding.reshape(-1, 2 * DIM)
    base2 = base_node_embedding.reshape(-1, 2 * DIM)
    s, na, nb, r = _sc_gather_dot(srcp, srch, ctxp, ctxh,
                                  purep, pureh, perp, perh,
                                  node2, noise2, base2)
    out = _finish(targets.reshape(128, 128), s.reshape(128, 128),
                  na.reshape(128, 128), nb.reshape(128, 128),
                  r.reshape(128, 128))
    return out.reshape(())


# speedup vs baseline: 1.0381x; 1.0170x over previous
import functools

import jax
import jax.numpy as jnp
from jax import lax
from jax.experimental import pallas as pl
from jax.experimental.pallas import tpu as pltpu
from jax.experimental.pallas import tpu_sc as plsc

DIM = 64
B = 16384
LAMBD = 0.1
NW = 32
CHUNK = B // NW
SEG = 128
NSEG = CHUNK // SEG
GROUPS = CHUNK // 16

_mesh = plsc.VectorSubcoreMesh(core_axis_name="c", subcore_axis_name="s",
                               num_cores=2, num_subcores=16)


@functools.partial(
    pl.kernel,
    mesh=_mesh,
    compiler_params=pltpu.CompilerParams(needs_layout_passes=False,
                                         use_tc_tiling_on_sc=False),
    out_type=[
        jax.ShapeDtypeStruct((B,), jnp.float32),
        jax.ShapeDtypeStruct((B,), jnp.float32),
        jax.ShapeDtypeStruct((B,), jnp.float32),
        jax.ShapeDtypeStruct((B,), jnp.float32),
    ],
    scratch_types=[
        pltpu.VMEM((NSEG, SEG), jnp.int32),
        pltpu.VMEM((NSEG, SEG), jnp.int32),
        pltpu.VMEM((CHUNK, DIM), jnp.float32),
        pltpu.VMEM((CHUNK, DIM), jnp.float32),
        pltpu.VMEM((CHUNK,), jnp.float32),
        pltpu.VMEM((CHUNK,), jnp.float32),
        pltpu.VMEM((CHUNK,), jnp.float32),
        pltpu.SemaphoreType.DMA,
    ],
)
def _sc_gather_dot(src_hbm, ctx_hbm, pure_hbm, per_hbm,
                   node_hbm, noise_hbm, base_hbm,
                   s_out, na_out, nb_out, r_out,
                   idx_a, idx_b, rows_a, rows_b, s_v, na_v, nb_v, sem):
    wid = lax.axis_index("s") * 2 + lax.axis_index("c")
    base = wid * CHUNK

    def gather_pair(tab_a, tab_b):
        handles = []
        for k in range(NSEG):
            handles.append(pltpu.async_copy(
                tab_a.at[idx_a.at[k]], rows_a.at[pl.ds(k * SEG, SEG)], sem))
            handles.append(pltpu.async_copy(
                tab_b.at[idx_b.at[k]], rows_b.at[pl.ds(k * SEG, SEG)], sem))
        for h in handles:
            h.wait()

    zero = jnp.zeros((16,), jnp.float32)

    pltpu.sync_copy(src_hbm.at[wid], idx_a)
    pltpu.sync_copy(ctx_hbm.at[wid], idx_b)
    gather_pair(node_hbm, noise_hbm)

    def main_group(g, _):
        rows = g * 16 + lax.iota(jnp.int32, 16)

        def col(j, acc):
            s, na, nb = acc
            cols = jnp.full((16,), 0, jnp.int32) + j
            a = plsc.load_gather(rows_a, [rows, cols])
            b = plsc.load_gather(rows_b, [rows, cols])
            return (s + a * b, na + a * a, nb + b * b)

        s, na, nb = lax.fori_loop(0, DIM, col, (zero, zero, zero))
        s_v[pl.ds(g * 16, 16)] = s
        na_v[pl.ds(g * 16, 16)] = na
        nb_v[pl.ds(g * 16, 16)] = nb
        return 0

    lax.fori_loop(0, GROUPS, main_group, 0)
    pltpu.sync_copy(s_v, s_out.at[pl.ds(base, CHUNK)])
    pltpu.sync_copy(na_v, na_out.at[pl.ds(base, CHUNK)])
    pltpu.sync_copy(nb_v, nb_out.at[pl.ds(base, CHUNK)])

    pltpu.sync_copy(pure_hbm.at[wid], idx_a)
    pltpu.sync_copy(per_hbm.at[wid], idx_b)
    gather_pair(node_hbm, base_hbm)

    def reg_group(g, _):
        rows = g * 16 + lax.iota(jnp.int32, 16)

        def col(j, s):
            cols = jnp.full((16,), 0, jnp.int32) + j
            a = plsc.load_gather(rows_a, [rows, cols])
            b = plsc.load_gather(rows_b, [rows, cols])
            return s + a * b

        s = lax.fori_loop(0, DIM, col, zero)
        s_v[pl.ds(g * 16, 16)] = s
        return 0

    lax.fori_loop(0, GROUPS, reg_group, 0)
    pltpu.sync_copy(s_v, r_out.at[pl.ds(base, CHUNK)])


def _finish_body(t_ref, s_ref, na_ref, nb_ref, r_ref, o_ref):
    na = jnp.maximum(jnp.sqrt(na_ref[...]), 1e-12)
    nb = jnp.maximum(jnp.sqrt(nb_ref[...]), 1e-12)
    scores = jax.nn.sigmoid(s_ref[...] / (na * nb))
    t = t_ref[...]
    main = t * jnp.log(scores) + (1.0 - t) * jnp.log(1.0 - scores)
    main_loss = -jnp.mean(main)
    r = jax.nn.sigmoid(jnp.clip(r_ref[...], -15.0, 15.0))
    reg_loss = -jnp.mean(jnp.log(r))
    o_ref[...] = jnp.reshape(main_loss + LAMBD * reg_loss, (1, 1))


_finish = pl.pallas_call(
    _finish_body,
    out_shape=jax.ShapeDtypeStruct((1, 1), jnp.float32),
)


@jax.jit
def kernel(sources, contexts, targets, personas, pure_sources,
           node_embedding, node_noise_embedding, base_node_embedding):
    src = sources.astype(jnp.int32).reshape(NW, NSEG, SEG)
    ctx = contexts.astype(jnp.int32).reshape(NW, NSEG, SEG)
    pure = pure_sources.astype(jnp.int32).reshape(NW, NSEG, SEG)
    per = personas.astype(jnp.int32).reshape(NW, NSEG, SEG)
    s, na, nb, r = _sc_gather_dot(src, ctx, pure, per,
                                  node_embedding, node_noise_embedding,
                                  base_node_embedding)
    out = _finish(targets.reshape(128, 128), s.reshape(128, 128),
                  na.reshape(128, 128), nb.reshape(128, 128),
                  r.reshape(128, 128))
    return out.reshape(())
